# baseline (speedup 1.0000x reference)
import jax
import jax.numpy as jnp
from jax.experimental import pallas as pl
from jax.experimental.pallas import tpu as pltpu

B, T, C = 16, 2048, 80
BT = 128
GRID = T // BT
NSLOTS = 6
STOP_WEIGHT = 8.0


def _start_copies(hbm_refs, bufs, sems, step, slot):
    for k, (hbm, buf) in enumerate(zip(hbm_refs, bufs)):
        pltpu.make_async_copy(
            hbm.at[:, pl.ds(step * BT, BT), :],
            buf.at[slot],
            sems.at[slot, k],
        ).start()


def _loss_body(len_ref, x_ref, pm_hbm, qm_hbm, ml_hbm, out_ref,
               pm_buf, qm_buf, ml_buf, sems):
    i = pl.program_id(0)
    hbm_refs = (pm_hbm, qm_hbm, ml_hbm)
    bufs = (pm_buf, qm_buf, ml_buf)

    @pl.when(i == 0)
    def _init():
        out_ref[0] = 0.0
        out_ref[1] = 0.0
        out_ref[2] = 0.0
        out_ref[3] = 0.0
        for s in range(NSLOTS - 1):
            _start_copies(hbm_refs, bufs, sems, s, s)

    nxt = i + NSLOTS - 1

    @pl.when(nxt < GRID)
    def _prefetch():
        _start_copies(hbm_refs, bufs, sems, nxt, nxt % NSLOTS)

    slot = i % NSLOTS
    for k, (hbm, buf) in enumerate(zip(hbm_refs, bufs)):
        pltpu.make_async_copy(
            hbm.at[:, pl.ds(i * BT, BT), :],
            buf.at[slot],
            sems.at[slot, k],
        ).wait()

    lens = len_ref[:, :1]

    t = jax.lax.broadcasted_iota(jnp.int32, (B, BT), 1) + i * BT
    m = jnp.where(t < lens, 1.0, 0.0)
    m3 = m[:, :, None]

    mel = ml_buf[slot]
    dp = pm_buf[slot] - mel
    dq = qm_buf[slot] - mel
    se_p = jnp.sum(dp * dp * m3)
    se_q = jnp.sum(dq * dq * m3)

    y = jnp.where(t == lens - 1, 1.0, 0.0)
    x = x_ref[...]
    sp_neg = jnp.maximum(-x, 0.0) + jnp.log1p(jnp.exp(-jnp.abs(x)))
    per = STOP_WEIGHT * y * sp_neg + (1.0 - y) * (x + sp_neg)
    s_stop = jnp.sum(per * m)
    s_n = jnp.sum(m)

    out_ref[0] += se_p
    out_ref[1] += se_q
    out_ref[2] += s_stop
    out_ref[3] += s_n

    @pl.when(i == GRID - 1)
    def _finish():
        n_valid = out_ref[3]
        pred_mel_loss = out_ref[0] / (n_valid * C)
        post_mel_loss = out_ref[1] / (n_valid * C)
        stop_loss = out_ref[2] / n_valid
        total = pred_mel_loss + 0.5 * post_mel_loss + stop_loss
        out_ref[0] = total
        out_ref[1] = pred_mel_loss
        out_ref[2] = post_mel_loss
        out_ref[3] = stop_loss


@jax.jit
def _ttsloss(pred_melspec, post_melspec, pred_stop, mel, lengths):
    len_b = jnp.broadcast_to(lengths.astype(jnp.int32)[:, None], (B, 128))

    out = pl.pallas_call(
        _loss_body,
        grid=(GRID,),
        in_specs=[
            pl.BlockSpec((B, 128), lambda i: (0, 0)),
            pl.BlockSpec((B, BT), lambda i: (0, i)),
            pl.BlockSpec(memory_space=pl.ANY),
            pl.BlockSpec(memory_space=pl.ANY),
            pl.BlockSpec(memory_space=pl.ANY),
        ],
        out_specs=pl.BlockSpec(memory_space=pltpu.SMEM),
        out_shape=jax.ShapeDtypeStruct((4,), jnp.float32),
        scratch_shapes=[
            pltpu.VMEM((NSLOTS, B, BT, C), jnp.float32),
            pltpu.VMEM((NSLOTS, B, BT, C), jnp.float32),
            pltpu.VMEM((NSLOTS, B, BT, C), jnp.float32),
            pltpu.SemaphoreType.DMA((NSLOTS, 3)),
        ],
    )(len_b, pred_stop, pred_melspec, post_melspec, mel)

    return out


def kernel(pred_melspec, post_melspec, pred_stop, mel, lengths):
    return _ttsloss(pred_melspec, post_melspec, pred_stop, mel, lengths)


# speedup vs baseline: 4.6349x; 4.6349x over previous
import jax
import jax.numpy as jnp
from jax.experimental import pallas as pl
from jax.experimental.pallas import tpu as pltpu

B, T, C = 16, 2048, 80
BT = 512
GRID = T // BT
STOP_WEIGHT = 8.0


def _loss_body(len_ref, pm_ref, qm_ref, ml_ref, x_ref, out_ref):
    i = pl.program_id(0)

    @pl.when(i == 0)
    def _init():
        out_ref[0] = 0.0
        out_ref[1] = 0.0
        out_ref[2] = 0.0
        out_ref[3] = 0.0

    lens = len_ref[:, :1]

    t2 = jax.lax.broadcasted_iota(jnp.int32, (B, BT), 1) + i * BT
    m2 = jnp.where(t2 < lens, 1.0, 0.0)
    m3 = m2[:, None, :]

    mel = ml_ref[...]
    dp = pm_ref[...] - mel
    dq = qm_ref[...] - mel
    se_p = jnp.sum(dp * dp * m3)
    se_q = jnp.sum(dq * dq * m3)

    y = jnp.where(t2 == lens - 1, 1.0, 0.0)
    x = x_ref[...]
    sp_neg = jnp.maximum(-x, 0.0) + jnp.log1p(jnp.exp(-jnp.abs(x)))
    per = STOP_WEIGHT * y * sp_neg + (1.0 - y) * (x + sp_neg)
    s_stop = jnp.sum(per * m2)
    s_n = jnp.sum(m2)

    out_ref[0] += se_p
    out_ref[1] += se_q
    out_ref[2] += s_stop
    out_ref[3] += s_n

    @pl.when(i == GRID - 1)
    def _finish():
        n_valid = out_ref[3]
        pred_mel_loss = out_ref[0] / (n_valid * C)
        post_mel_loss = out_ref[1] / (n_valid * C)
        stop_loss = out_ref[2] / n_valid
        total = pred_mel_loss + 0.5 * post_mel_loss + stop_loss
        out_ref[0] = total
        out_ref[1] = pred_mel_loss
        out_ref[2] = post_mel_loss
        out_ref[3] = stop_loss


@jax.jit
def _ttsloss(pred_melspec, post_melspec, pred_stop, mel, lengths):
    pm = jnp.transpose(pred_melspec, (0, 2, 1))
    qm = jnp.transpose(post_melspec, (0, 2, 1))
    ml = jnp.transpose(mel, (0, 2, 1))
    len_b = jnp.broadcast_to(lengths.astype(jnp.int32)[:, None], (B, 128))

    out = pl.pallas_call(
        _loss_body,
        grid=(GRID,),
        in_specs=[
            pl.BlockSpec((B, 128), lambda i: (0, 0)),
            pl.BlockSpec((B, C, BT), lambda i: (0, 0, i)),
            pl.BlockSpec((B, C, BT), lambda i: (0, 0, i)),
            pl.BlockSpec((B, C, BT), lambda i: (0, 0, i)),
            pl.BlockSpec((B, BT), lambda i: (0, i)),
        ],
        out_specs=pl.BlockSpec(memory_space=pltpu.SMEM),
        out_shape=jax.ShapeDtypeStruct((4,), jnp.float32),
    )(len_b, pm, qm, ml, pred_stop)

    return out


def kernel(pred_melspec, post_melspec, pred_stop, mel, lengths):
    return _ttsloss(pred_melspec, post_melspec, pred_stop, mel, lengths)
